# Initial kernel scaffold; baseline (speedup 1.0000x reference)
import functools

import jax
import jax.numpy as jnp
from jax import lax
from jax.experimental import pallas as pl
from jax.experimental.pallas import tpu as pltpu
from jax.experimental.pallas import tpu_sc as plsc

EMB_DIM = 64
SCALE = 8.0
LANES = 16
CHUNK = 128


@functools.lru_cache(maxsize=None)
def _build(n_rows: int):
    info = plsc.get_sparse_core_info()
    nc, ns = info.num_cores, info.num_subcores
    nw = nc * ns
    assert n_rows % nw == 0
    n_per_w = n_rows // nw
    assert n_per_w % CHUNK == 0
    n_chunks = n_per_w // CHUNK

    mesh = plsc.VectorSubcoreMesh(core_axis_name="c", subcore_axis_name="s")

    @functools.partial(
        pl.kernel,
        out_type=jax.ShapeDtypeStruct((n_rows, EMB_DIM), jnp.float32),
        scratch_types=[
            pltpu.VMEM((CHUNK,), jnp.int32),
            pltpu.VMEM((CHUNK,), jnp.int32),
            pltpu.VMEM((CHUNK, EMB_DIM), jnp.float32),
            pltpu.VMEM((CHUNK, EMB_DIM), jnp.float32),
            pltpu.SemaphoreType.DMA,
            pltpu.SemaphoreType.DMA,
        ],
        mesh=mesh,
    )
    def fused(word_hbm, pos_hbm, emb0_hbm, emb1_hbm, out_hbm,
              widx, pidx, g0, g1, sem0, sem1):
        wid = lax.axis_index("s") * nc + lax.axis_index("c")
        w_base = wid * n_per_w

        def chunk_body(ci, carry):
            base = w_base + ci * CHUNK
            pltpu.sync_copy(word_hbm.at[pl.ds(base, CHUNK)], widx)
            pltpu.sync_copy(pos_hbm.at[pl.ds(base, CHUNK)], pidx)
            cp0 = pltpu.async_copy(emb0_hbm.at[widx], g0, sem0)
            cp1 = pltpu.async_copy(emb1_hbm.at[pidx], g1, sem1)
            cp0.wait()
            cp1.wait()

            def row_body(r, rcarry):
                for j in range(EMB_DIM // LANES):
                    sl = pl.ds(j * LANES, LANES)
                    g0[r, sl] = g0[r, sl] * SCALE + g1[r, sl]
                return rcarry

            lax.fori_loop(0, CHUNK, row_body, 0, unroll=2)
            pltpu.sync_copy(g0, out_hbm.at[pl.ds(base, CHUNK)])
            return carry

        lax.fori_loop(0, n_chunks, chunk_body, 0)

    return fused


def kernel(src_word, src_pos, emb0_weight, emb1_weight):
    b, s = src_word.shape
    n_rows = b * s
    word = src_word.reshape(n_rows).astype(jnp.int32)
    pos = src_pos.reshape(n_rows).astype(jnp.int32)
    out = _build(n_rows)(word, pos, emb0_weight, emb1_weight)
    return out.reshape(b, s, EMB_DIM)


# speedup vs baseline: 1.0734x; 1.0734x over previous
import functools

import jax
import jax.numpy as jnp
from jax import lax
from jax.experimental import pallas as pl
from jax.experimental.pallas import tpu as pltpu
from jax.experimental.pallas import tpu_sc as plsc

EMB_DIM = 64
SCALE = 8.0
LANES = 16
CHUNK = 128


@functools.lru_cache(maxsize=None)
def _build(n_rows: int):
    info = plsc.get_sparse_core_info()
    nc, ns = info.num_cores, info.num_subcores
    nw = nc * ns
    assert n_rows % nw == 0
    n_per_w = n_rows // nw
    assert n_per_w % CHUNK == 0
    n_chunks = n_per_w // CHUNK

    mesh = plsc.VectorSubcoreMesh(core_axis_name="c", subcore_axis_name="s")

    @functools.partial(
        pl.kernel,
        out_type=jax.ShapeDtypeStruct((n_rows, EMB_DIM), jnp.float32),
        scratch_types=[
            pltpu.VMEM((CHUNK,), jnp.int32),
            pltpu.VMEM((CHUNK,), jnp.int32),
            pltpu.VMEM((CHUNK, EMB_DIM), jnp.float32),
            pltpu.VMEM((CHUNK, EMB_DIM), jnp.float32),
            pltpu.SemaphoreType.DMA,
            pltpu.SemaphoreType.DMA,
        ],
        mesh=mesh,
        compiler_params=pltpu.CompilerParams(use_tc_tiling_on_sc=False),
    )
    def fused(word_hbm, pos_hbm, emb0_hbm, emb1_hbm, out_hbm,
              widx, pidx, g0, g1, sem0, sem1):
        wid = lax.axis_index("s") * nc + lax.axis_index("c")
        w_base = wid * n_per_w

        def chunk_body(ci, carry):
            base = w_base + ci * CHUNK
            pltpu.sync_copy(word_hbm.at[pl.ds(base, CHUNK)], widx)
            pltpu.sync_copy(pos_hbm.at[pl.ds(base, CHUNK)], pidx)
            cp0 = pltpu.async_copy(emb0_hbm.at[widx], g0, sem0)
            cp1 = pltpu.async_copy(emb1_hbm.at[pidx], g1, sem1)
            cp0.wait()
            cp1.wait()

            def row_body(r, rcarry):
                for j in range(EMB_DIM // LANES):
                    sl = pl.ds(j * LANES, LANES)
                    g0[r, sl] = g0[r, sl] * SCALE + g1[r, sl]
                return rcarry

            lax.fori_loop(0, CHUNK, row_body, 0, unroll=2)
            pltpu.sync_copy(g0, out_hbm.at[pl.ds(base, CHUNK)])
            return carry

        lax.fori_loop(0, n_chunks, chunk_body, 0)

    return fused


def kernel(src_word, src_pos, emb0_weight, emb1_weight):
    b, s = src_word.shape
    n_rows = b * s
    word = src_word.reshape(n_rows).astype(jnp.int32)
    pos = src_pos.reshape(n_rows).astype(jnp.int32)
    out = _build(n_rows)(word, pos, emb0_weight, emb1_weight)
    return out.reshape(b, s, EMB_DIM)


# speedup vs baseline: 1.1685x; 1.0886x over previous
import functools

import jax
import jax.numpy as jnp
from jax import lax
from jax.experimental import pallas as pl
from jax.experimental.pallas import tpu as pltpu
from jax.experimental.pallas import tpu_sc as plsc

EMB_DIM = 64
SCALE = 8.0
LANES = 16
CHUNK = 128


@functools.lru_cache(maxsize=None)
def _build(n_rows: int):
    info = plsc.get_sparse_core_info()
    nc, ns = info.num_cores, info.num_subcores
    nw = nc * ns
    assert n_rows % nw == 0
    n_per_w = n_rows // nw
    assert n_per_w % (2 * CHUNK) == 0
    n_chunks = n_per_w // CHUNK
    n_pairs = n_chunks // 2

    mesh = plsc.VectorSubcoreMesh(core_axis_name="c", subcore_axis_name="s")

    @functools.partial(
        pl.kernel,
        out_type=jax.ShapeDtypeStruct((n_rows, EMB_DIM), jnp.float32),
        scratch_types=[
            pltpu.VMEM((n_per_w,), jnp.int32),
            pltpu.VMEM((n_per_w,), jnp.int32),
            pltpu.VMEM((CHUNK, EMB_DIM), jnp.float32),
            pltpu.VMEM((CHUNK, EMB_DIM), jnp.float32),
            pltpu.VMEM((CHUNK, EMB_DIM), jnp.float32),
            pltpu.VMEM((CHUNK, EMB_DIM), jnp.float32),
            pltpu.SemaphoreType.DMA,
            pltpu.SemaphoreType.DMA,
            pltpu.SemaphoreType.DMA,
            pltpu.SemaphoreType.DMA,
        ],
        mesh=mesh,
        compiler_params=pltpu.CompilerParams(use_tc_tiling_on_sc=False),
    )
    def fused(word_hbm, pos_hbm, emb0_hbm, emb1_hbm, out_hbm,
              widx, pidx, g0a, g1a, g0b, g1b, sga, sgb, swa, swb):
        wid = lax.axis_index("s") * nc + lax.axis_index("c")
        w_base = wid * n_per_w

        pltpu.sync_copy(word_hbm.at[pl.ds(w_base, n_per_w)], widx)
        pltpu.sync_copy(pos_hbm.at[pl.ds(w_base, n_per_w)], pidx)

        def fire(ci, g0, g1, sem):
            isl = pl.ds(ci * CHUNK, CHUNK)
            return (pltpu.async_copy(emb0_hbm.at[widx.at[isl]], g0, sem),
                    pltpu.async_copy(emb1_hbm.at[pidx.at[isl]], g1, sem))

        def wait_gathers(g0, g1, sem):
            pltpu.make_async_copy(emb0_hbm.at[pl.ds(0, CHUNK)], g0, sem).wait()
            pltpu.make_async_copy(emb1_hbm.at[pl.ds(0, CHUNK)], g1, sem).wait()

        def compute(g0, g1):
            def row_body(r, rcarry):
                for j in range(EMB_DIM // LANES):
                    sl = pl.ds(j * LANES, LANES)
                    g0[r, sl] = g0[r, sl] * SCALE + g1[r, sl]
                return rcarry
            lax.fori_loop(0, CHUNK, row_body, 0, unroll=4)

        def wb(ci, g0, sem):
            base = w_base + ci * CHUNK
            return pltpu.async_copy(g0, out_hbm.at[pl.ds(base, CHUNK)], sem)

        def wait_wb(g0, sem):
            pltpu.make_async_copy(g0, out_hbm.at[pl.ds(0, CHUNK)], sem).wait()

        fire(0, g0a, g1a, sga)
        fire(1, g0b, g1b, sgb)

        def pair_body(p, carry):
            ci = 2 * p
            wait_gathers(g0a, g1a, sga)
            compute(g0a, g1a)
            wb(ci, g0a, swa)
            wait_gathers(g0b, g1b, sgb)
            compute(g0b, g1b)
            wb(ci + 1, g0b, swb)
            nxt_a = lax.min(ci + 2, n_chunks - 1)
            nxt_b = lax.min(ci + 3, n_chunks - 1)
            wait_wb(g0a, swa)
            fire(nxt_a, g0a, g1a, sga)
            wait_wb(g0b, swb)
            fire(nxt_b, g0b, g1b, sgb)
            return carry

        lax.fori_loop(0, n_pairs, pair_body, 0)

        wait_gathers(g0a, g1a, sga)
        wait_gathers(g0b, g1b, sgb)

    return fused


def kernel(src_word, src_pos, emb0_weight, emb1_weight):
    b, s = src_word.shape
    n_rows = b * s
    word = src_word.reshape(n_rows).astype(jnp.int32)
    pos = src_pos.reshape(n_rows).astype(jnp.int32)
    out = _build(n_rows)(word, pos, emb0_weight, emb1_weight)
    return out.reshape(b, s, EMB_DIM)


# speedup vs baseline: 1.2674x; 1.0846x over previous
import functools

import jax
import jax.numpy as jnp
from jax import lax
from jax.experimental import pallas as pl
from jax.experimental.pallas import tpu as pltpu
from jax.experimental.pallas import tpu_sc as plsc

EMB_DIM = 64
SCALE = 8.0
LANES = 16
CHUNK = 128
GROUPS = CHUNK // LANES


@functools.lru_cache(maxsize=None)
def _build(n_rows: int, n_pos: int):
    info = plsc.get_sparse_core_info()
    nc, ns = info.num_cores, info.num_subcores
    nw = nc * ns
    assert n_rows % nw == 0
    n_per_w = n_rows // nw
    assert n_per_w % (2 * CHUNK) == 0
    n_chunks = n_per_w // CHUNK
    n_pairs = n_chunks // 2

    mesh = plsc.VectorSubcoreMesh(core_axis_name="c", subcore_axis_name="s")

    @functools.partial(
        pl.kernel,
        out_type=jax.ShapeDtypeStruct((n_rows, EMB_DIM), jnp.float32),
        scratch_types=[
            pltpu.VMEM((n_per_w,), jnp.int32),
            pltpu.VMEM((n_per_w,), jnp.int32),
            pltpu.VMEM((n_per_w,), jnp.int32),
            pltpu.VMEM((n_pos, EMB_DIM), jnp.float32),
            pltpu.VMEM((CHUNK, 2 * EMB_DIM), jnp.float32),
            pltpu.VMEM((CHUNK, 2 * EMB_DIM), jnp.float32),
            pltpu.VMEM((CHUNK, EMB_DIM), jnp.float32),
            pltpu.VMEM((CHUNK, EMB_DIM), jnp.float32),
            pltpu.SemaphoreType.DMA,
            pltpu.SemaphoreType.DMA,
            pltpu.SemaphoreType.DMA,
            pltpu.SemaphoreType.DMA,
        ],
        mesh=mesh,
        compiler_params=pltpu.CompilerParams(use_tc_tiling_on_sc=True),
    )
    def fused(word_hbm, pos_hbm, pair0_hbm, emb1_hbm, out_hbm,
              wpair, woff, pidx, e1, ga, gb, oa, ob, sga, sgb, swa, swb):
        wid = lax.axis_index("s") * nc + lax.axis_index("c")
        w_base = wid * n_per_w

        pltpu.sync_copy(word_hbm.at[pl.ds(w_base, n_per_w)], wpair)
        pltpu.sync_copy(pos_hbm.at[pl.ds(w_base, n_per_w)], pidx)
        pltpu.sync_copy(emb1_hbm, e1)

        def split_body(i, carry):
            sl = pl.ds(i * LANES, LANES)
            w = wpair[sl]
            woff[sl] = (w & 1) * EMB_DIM
            wpair[sl] = lax.shift_right_logical(w, 1)
            return carry

        lax.fori_loop(0, n_per_w // LANES, split_body, 0, unroll=4)

        def fire(ci, g, sem):
            isl = pl.ds(ci * CHUNK, CHUNK)
            return pltpu.async_copy(pair0_hbm.at[wpair.at[isl]], g, sem)

        def wait_gather(g, sem):
            pltpu.make_async_copy(pair0_hbm.at[pl.ds(0, CHUNK)], g, sem).wait()

        def compute(ci, g, o):
            cbase = ci * CHUNK

            def group_body(gi, carry):
                rbase = gi * LANES
                sl = pl.ds(cbase + rbase, LANES)
                wo = woff[sl]
                pv = pidx[sl]
                for r in range(LANES):
                    wo_s = wo[r]
                    p_s = pv[r]
                    for j in range(EMB_DIM // LANES):
                        csl = pl.ds(j * LANES, LANES)
                        a = g[rbase + r, pl.ds(wo_s + j * LANES, LANES)]
                        b = e1[p_s, csl]
                        o[rbase + r, csl] = a * SCALE + b
                return carry

            lax.fori_loop(0, GROUPS, group_body, 0)

        def wb(ci, o, sem):
            base = w_base + ci * CHUNK
            return pltpu.async_copy(o, out_hbm.at[pl.ds(base, CHUNK)], sem)

        def wait_wb(o, sem):
            pltpu.make_async_copy(o, out_hbm.at[pl.ds(0, CHUNK)], sem).wait()

        fire(0, ga, sga)
        fire(1, gb, sgb)

        def pair_body(p, carry):
            ci = 2 * p
            wait_gather(ga, sga)
            compute(ci, ga, oa)
            fire(lax.min(ci + 2, n_chunks - 1), ga, sga)
            wb(ci, oa, swa)
            wait_gather(gb, sgb)
            compute(ci + 1, gb, ob)
            fire(lax.min(ci + 3, n_chunks - 1), gb, sgb)
            wb(ci + 1, ob, swb)
            wait_wb(oa, swa)
            wait_wb(ob, swb)
            return carry

        lax.fori_loop(0, n_pairs, pair_body, 0)

        wait_gather(ga, sga)
        wait_gather(gb, sgb)

    return fused


def kernel(src_word, src_pos, emb0_weight, emb1_weight):
    b, s = src_word.shape
    n_rows = b * s
    n_vocab, d = emb0_weight.shape
    n_pos = emb1_weight.shape[0]
    word = src_word.reshape(n_rows).astype(jnp.int32)
    pos = src_pos.reshape(n_rows).astype(jnp.int32)
    pair0 = emb0_weight.reshape(n_vocab // 2, 2 * d)
    out = _build(n_rows, n_pos)(word, pos, pair0, emb1_weight)
    return out.reshape(b, s, EMB_DIM)


# speedup vs baseline: 1.8387x; 1.4507x over previous
import functools

import jax
import jax.numpy as jnp
from jax import lax
from jax.experimental import pallas as pl
from jax.experimental.pallas import tpu as pltpu
from jax.experimental.pallas import tpu_sc as plsc

EMB_DIM = 64
SCALE = 8.0
LANES = 16
CHUNK = 128
GROUPS = CHUNK // LANES


@functools.lru_cache(maxsize=None)
def _build(n_rows: int, n_pos: int):
    info = plsc.get_sparse_core_info()
    nc, ns = info.num_cores, info.num_subcores
    nw = nc * ns
    assert n_rows % nw == 0
    n_per_w = n_rows // nw
    assert n_per_w % (2 * CHUNK) == 0
    n_chunks = n_per_w // CHUNK
    n_pairs = n_chunks // 2

    mesh = plsc.VectorSubcoreMesh(core_axis_name="c", subcore_axis_name="s")

    @functools.partial(
        pl.kernel,
        out_type=jax.ShapeDtypeStruct((n_rows, EMB_DIM), jnp.float32),
        scratch_types=[
            pltpu.VMEM((n_per_w,), jnp.int32),
            pltpu.VMEM((n_per_w,), jnp.int32),
            pltpu.VMEM((n_pos, EMB_DIM), jnp.float32),
            pltpu.VMEM((CHUNK, EMB_DIM), jnp.float32),
            pltpu.VMEM((CHUNK, EMB_DIM), jnp.float32),
            pltpu.VMEM((CHUNK, EMB_DIM), jnp.float32),
            pltpu.VMEM((CHUNK, EMB_DIM), jnp.float32),
            pltpu.SemaphoreType.DMA,
            pltpu.SemaphoreType.DMA,
            pltpu.SemaphoreType.DMA,
            pltpu.SemaphoreType.DMA,
        ],
        mesh=mesh,
        compiler_params=pltpu.CompilerParams(use_tc_tiling_on_sc=True),
    )
    def fused(word_hbm, pos_hbm, table_hbm, emb1_hbm, out_hbm,
              widx, pidx, e1, ga, gb, oa, ob, sga, sgb, swa, swb):
        wid = lax.axis_index("s") * nc + lax.axis_index("c")
        w_base = wid * n_per_w

        pltpu.sync_copy(word_hbm.at[pl.ds(w_base, n_per_w)], widx)
        pltpu.sync_copy(pos_hbm.at[pl.ds(w_base, n_per_w)], pidx)
        pltpu.sync_copy(emb1_hbm, e1)

        def fire(ci, g, sem):
            cbase = ci * CHUNK

            def grp(gi, carry):
                wv = widx[pl.ds(cbase + gi * LANES, LANES)]
                for r in range(LANES):
                    i_s = wv[r]
                    pltpu.async_copy(table_hbm.at[i_s], g.at[gi * LANES + r], sem)
                return carry

            lax.fori_loop(0, GROUPS, grp, 0)

        def wait_gather(g, sem):
            pltpu.make_async_copy(out_hbm.at[pl.ds(0, CHUNK)], g, sem).wait()

        def compute(ci, g, o):
            cbase = ci * CHUNK

            def group_body(gi, carry):
                rbase = gi * LANES
                pv = pidx[pl.ds(cbase + rbase, LANES)]
                for r in range(LANES):
                    p_s = pv[r]
                    for j in range(EMB_DIM // LANES):
                        csl = pl.ds(j * LANES, LANES)
                        o[rbase + r, csl] = g[rbase + r, csl] * SCALE + e1[p_s, csl]
                return carry

            lax.fori_loop(0, GROUPS, group_body, 0)

        def wb(ci, o, sem):
            base = w_base + ci * CHUNK
            return pltpu.async_copy(o, out_hbm.at[pl.ds(base, CHUNK)], sem)

        def wait_wb(o, sem):
            pltpu.make_async_copy(o, out_hbm.at[pl.ds(0, CHUNK)], sem).wait()

        fire(0, ga, sga)
        fire(1, gb, sgb)

        def pair_body(p, carry):
            ci = 2 * p
            wait_gather(ga, sga)
            compute(ci, ga, oa)
            fire(lax.min(ci + 2, n_chunks - 1), ga, sga)
            wb(ci, oa, swa)
            wait_gather(gb, sgb)
            compute(ci + 1, gb, ob)
            fire(lax.min(ci + 3, n_chunks - 1), gb, sgb)
            wb(ci + 1, ob, swb)
            wait_wb(oa, swa)
            wait_wb(ob, swb)
            return carry

        lax.fori_loop(0, n_pairs, pair_body, 0)

        wait_gather(ga, sga)
        wait_gather(gb, sgb)

    return fused


def kernel(src_word, src_pos, emb0_weight, emb1_weight):
    b, s = src_word.shape
    n_rows = b * s
    n_pos = emb1_weight.shape[0]
    word = src_word.reshape(n_rows).astype(jnp.int32)
    pos = src_pos.reshape(n_rows).astype(jnp.int32)
    out = _build(n_rows, n_pos)(word, pos, emb0_weight, emb1_weight)
    return out.reshape(b, s, EMB_DIM)


# speedup vs baseline: 2.3562x; 1.2815x over previous
import functools

import jax
import jax.numpy as jnp
from jax import lax
from jax.experimental import pallas as pl
from jax.experimental.pallas import tpu as pltpu
from jax.experimental.pallas import tpu_sc as plsc

EMB_DIM = 64
SCALE = 8.0
LANES = 16
CHUNK = 128
GROUPS = CHUNK // LANES


@functools.lru_cache(maxsize=None)
def _build(n_rows: int, n_pos: int):
    info = plsc.get_sparse_core_info()
    nc, ns = info.num_cores, info.num_subcores
    nw = nc * ns
    assert n_rows % nw == 0
    n_per_w = n_rows // nw
    assert n_per_w % (2 * CHUNK) == 0
    n_chunks = n_per_w // CHUNK
    n_pairs = n_chunks // 2

    mesh = plsc.VectorSubcoreMesh(core_axis_name="c", subcore_axis_name="s")

    @functools.partial(
        pl.kernel,
        out_type=jax.ShapeDtypeStruct((n_rows, EMB_DIM), jnp.float32),
        scratch_types=[
            pltpu.VMEM((n_per_w,), jnp.int32),
            pltpu.VMEM((n_per_w,), jnp.int32),
            pltpu.VMEM((n_pos, EMB_DIM), jnp.float32),
            pltpu.VMEM((CHUNK, EMB_DIM), jnp.float32),
            pltpu.VMEM((CHUNK, EMB_DIM), jnp.float32),
            pltpu.VMEM((CHUNK, EMB_DIM), jnp.float32),
            pltpu.VMEM((CHUNK, EMB_DIM), jnp.float32),
            pltpu.SemaphoreType.DMA,
            pltpu.SemaphoreType.DMA,
            pltpu.SemaphoreType.DMA,
            pltpu.SemaphoreType.DMA,
        ],
        mesh=mesh,
        compiler_params=pltpu.CompilerParams(use_tc_tiling_on_sc=True),
    )
    def fused(word_hbm, pos_hbm, table_hbm, emb1_hbm, out_hbm,
              widx, pidx, e1, ga, gb, oa, ob, sga, sgb, swa, swb):
        wid = lax.axis_index("s") * nc + lax.axis_index("c")
        w_base = wid * n_per_w

        pltpu.sync_copy(word_hbm.at[pl.ds(w_base, n_per_w)], widx)
        pltpu.sync_copy(pos_hbm.at[pl.ds(w_base, n_per_w)], pidx)
        pltpu.sync_copy(emb1_hbm, e1)

        def fire(ci, g, sem):
            cbase = ci * CHUNK

            def grp(gi, carry):
                wv = widx[pl.ds(cbase + gi * LANES, LANES)]
                hi = lax.shift_right_logical(wv, 3)
                lo = wv & 7
                for r in range(LANES):
                    pltpu.async_copy(table_hbm.at[hi[r], lo[r]],
                                     g.at[gi * LANES + r], sem)
                return carry

            lax.fori_loop(0, GROUPS, grp, 0)

        def wait_gather(g, sem):
            pltpu.make_async_copy(out_hbm.at[pl.ds(0, CHUNK)], g, sem).wait()

        def compute(ci, g, o):
            cbase = ci * CHUNK

            def group_body(gi, carry):
                rbase = gi * LANES
                pv = pidx[pl.ds(cbase + rbase, LANES)]
                for r in range(LANES):
                    p_s = pv[r]
                    for j in range(EMB_DIM // LANES):
                        csl = pl.ds(j * LANES, LANES)
                        o[rbase + r, csl] = g[rbase + r, csl] * SCALE + e1[p_s, csl]
                return carry

            lax.fori_loop(0, GROUPS, group_body, 0)

        def wb(ci, o, sem):
            base = w_base + ci * CHUNK
            return pltpu.async_copy(o, out_hbm.at[pl.ds(base, CHUNK)], sem)

        def wait_wb(o, sem):
            pltpu.make_async_copy(o, out_hbm.at[pl.ds(0, CHUNK)], sem).wait()

        fire(0, ga, sga)
        fire(1, gb, sgb)

        def pair_body(p, carry):
            ci = 2 * p
            wait_gather(ga, sga)
            compute(ci, ga, oa)
            fire(lax.min(ci + 2, n_chunks - 1), ga, sga)
            wb(ci, oa, swa)
            wait_gather(gb, sgb)
            compute(ci + 1, gb, ob)
            fire(lax.min(ci + 3, n_chunks - 1), gb, sgb)
            wb(ci + 1, ob, swb)
            wait_wb(oa, swa)
            wait_wb(ob, swb)
            return carry

        lax.fori_loop(0, n_pairs, pair_body, 0)

        wait_gather(ga, sga)
        wait_gather(gb, sgb)

    return fused


def kernel(src_word, src_pos, emb0_weight, emb1_weight):
    b, s = src_word.shape
    n_rows = b * s
    n_pos = emb1_weight.shape[0]
    n_vocab, d = emb0_weight.shape
    word = src_word.reshape(n_rows).astype(jnp.int32)
    pos = src_pos.reshape(n_rows).astype(jnp.int32)
    table3 = emb0_weight.reshape(n_vocab // 8, 8, d)
    out = _build(n_rows, n_pos)(word, pos, table3, emb1_weight)
    return out.reshape(b, s, EMB_DIM)
